# baseline (speedup 1.0000x reference)
import jax
import jax.numpy as jnp
from jax import lax
from jax.experimental import pallas as pl
from jax.experimental.pallas import tpu as pltpu
from jax.experimental.pallas import tpu_sc as plsc

UNITS = 128
BLK = 128
SCK = 256
NW = 32


def _sc_body(table_hbm, idx_hbm, out_hbm, table_sp,
             src0, src1, src2, cidx, offbuf, pbuf, powbuf, biglist,
             idxb0, idxb1, si0, si1, ss0, ss1, ss2):
    srcs = (src0, src1, src2)
    ssem = (ss0, ss1, ss2)

    info = plsc.get_sparse_core_info()
    nc, ns = info.num_cores, info.num_subcores
    nw = nc * ns
    cid = lax.axis_index("c")
    sid = lax.axis_index("s")
    wid = sid * nc + cid

    @pl.when(sid == 0)
    def _():
        pltpu.sync_copy(table_hbm, table_sp)

    plsc.subcore_barrier()

    n_rows = out_hbm.shape[0]
    rows_per_w = n_rows // nw
    n_sck = rows_per_w // SCK
    n_pairs = n_sck // 2
    nl = rows_per_w // BLK
    base = wid * rows_per_w

    iota = lax.iota(jnp.int32, 16)
    shift_src = [jnp.maximum(iota - k, 0) for k in (1, 2, 4, 8)]
    shift_msk = [iota >= k for k in (1, 2, 4, 8)]
    lane15 = jnp.full((16,), 15, jnp.int32)
    zvec = jnp.zeros((16,), jnp.int32)
    ones = jnp.full((16,), 1, jnp.int32)
    f31 = jnp.full((16,), 31, jnp.int32)
    iota5 = jnp.minimum(iota * 5, f31 - 1)

    def prefix_incl(x):
        for st, (s, mk) in enumerate(zip(shift_src, shift_msk)):
            sub = pbuf.at[pl.ds(16 * st, 16)]
            sub[pl.ds(0, 16)] = x
            x = x + jnp.where(mk, plsc.load_gather(sub, [s]), zvec)
        return x

    def lane15_bcast(x):
        sub = pbuf.at[pl.ds(64, 16)]
        sub[pl.ds(0, 16)] = x
        return plsc.load_gather(sub, [lane15])

    for v in range(3):
        for k in range(8):
            cidx[pl.ds(16 * k, 16)] = jnp.full((16,), v, jnp.int32)
        pltpu.async_copy(table_sp.at[cidx], srcs[v], ssem[v]).wait()

    powbuf[pl.ds(0, 16)] = jnp.where(iota5 < f31, ones << iota5, ones)
    offbuf[pl.ds(0, 16)] = iota * rows_per_w

    def prefetch(sc, buf, sem):
        return pltpu.async_copy(idx_hbm.at[pl.ds(base + sc * SCK, SCK)], buf, sem)

    def wait_prefetch(buf, sem):
        pltpu.make_async_copy(idx_hbm.at[pl.ds(base, SCK)], buf, sem).wait()

    def scatter_block(v, j):
        return pltpu.async_copy(srcs[v], out_hbm.at[biglist.at[v * nl + j]],
                                ssem[v])

    def wait_scatter(v):
        pltpu.make_async_copy(srcs[v], out_hbm.at[biglist.at[0]], ssem[v]).wait()

    prefetch(0, idxb0, si0)
    prefetch(1, idxb1, si1)

    def do_superchunk(sc, buf):
        off_all = offbuf[pl.ds(0, 16)]
        for k in range(SCK // 16):
            idx16 = buf[pl.ds(16 * k, 16)]
            pos16 = (base + sc * SCK + 16 * k) + iota
            enc = plsc.load_gather(powbuf, [idx16])
            pf = prefix_incl(enc)
            rank = ((pf >> (idx16 * 5)) & f31) - ones
            offsel = plsc.load_gather(offbuf, [idx16])
            slot = offsel + rank
            plsc.store_scatter(biglist, [slot >> 7, slot & 127], pos16)
            tot = lane15_bcast(pf)
            off_all = off_all + ((tot >> iota5) & f31)
            offbuf[pl.ds(0, 16)] = off_all

    def flush(v, done):
        nb = (offbuf[pl.ds(0, 16)][v] - v * rows_per_w) >> 7

        def issue(j, c):
            scatter_block(v, j)
            return c

        lax.fori_loop(done, nb, issue, 0)
        return nb

    def pair(g, carry):
        dones = list(carry)
        for half in range(2):
            sc = 2 * g + half
            buf = (idxb0, idxb1)[half]
            sem = (si0, si1)[half]
            wait_prefetch(buf, sem)
            do_superchunk(sc, buf)

            @pl.when(sc + 2 < n_sck)
            def _():
                prefetch(sc + 2, buf, sem)

            for v in range(3):
                dones[v] = flush(v, dones[v])
        return tuple(dones)

    zs = jnp.zeros((), jnp.int32)
    d0, d1, d2 = lax.fori_loop(0, n_pairs, pair, (zs, zs, zs))

    for v, dv in ((0, d0), (1, d1), (2, d2)):
        cnt = offbuf[pl.ds(0, 16)][v] - v * rows_per_w
        rem = cnt & 127
        nb = cnt >> 7
        gr = v * nl + nb

        @pl.when(rem != 0)
        def _():
            padvec = plsc.load_gather(biglist,
                                      [jnp.full((16,), v * nl, jnp.int32), zvec])
            for k in range(8):
                cur = biglist[gr, pl.ds(16 * k, 16)]
                keep = (iota + 16 * k) < rem
                biglist[gr, pl.ds(16 * k, 16)] = jnp.where(keep, cur, padvec)
            scatter_block(v, nb)

        def drain(j, c):
            wait_scatter(v)
            return c

        n_drain = dv + jnp.where(rem != 0, 1, 0).astype(jnp.int32)
        lax.fori_loop(0, n_drain, drain, 0)


def kernel(inputs, w1, w2):
    B, T = inputs.shape
    U = w1.shape[1]
    n = B * T
    idx = inputs.reshape(-1).astype(jnp.int32)
    table = jnp.concatenate([jnp.zeros_like(w1), w1, w2], axis=0)
    mesh = plsc.VectorSubcoreMesh(core_axis_name="c", subcore_axis_name="s")
    rows_per_w = n // NW
    nl = rows_per_w // BLK
    k = pl.kernel(
        _sc_body,
        out_type=jax.ShapeDtypeStruct((n, U), jnp.float32),
        mesh=mesh,
        compiler_params=pltpu.CompilerParams(needs_layout_passes=False),
        scratch_types=(
            [pltpu.VMEM_SHARED((3, U), jnp.float32)]
            + [pltpu.VMEM((BLK, U), jnp.float32)] * 3
            + [pltpu.VMEM((BLK,), jnp.int32)]
            + [pltpu.VMEM((16,), jnp.int32)]
            + [pltpu.VMEM((80,), jnp.int32)]
            + [pltpu.VMEM((16,), jnp.int32)]
            + [pltpu.VMEM((3 * nl, BLK), jnp.int32)]
            + [pltpu.VMEM((SCK,), jnp.int32)] * 2
            + [pltpu.SemaphoreType.DMA] * 5
        ),
    )
    out = k(table, idx)
    return out.reshape(B, T, U)
